# baseline (speedup 1.0000x reference)
import functools

import jax
import jax.numpy as jnp
from jax.experimental import pallas as pl
from jax.experimental.pallas import tpu as pltpu


def _body(l_ref, lu_ref, ld_ref, x_ref, pW_ref, sW_ref, uW_ref, ua1_ref,
          ua2_ref, dW_ref, da1_ref, da2_ref, out_ref,
          fp_s, fu_s, fd_s, hs_s, s1u_s, s2u_s, s1d_s, s2d_s,
          accp, accu, accd, denu, dend, *, bm, bn):
    i = pl.program_id(0)
    j = pl.program_id(1)
    f32 = jnp.float32
    bf16 = jnp.bfloat16
    row = pl.ds(i * bm, bm)
    col = pl.ds(j * bn, bn)

    @pl.when(i == 0)
    def _project():
        xj = x_ref[col, :]
        fp_s[col, :] = jnp.dot(xj, pW_ref[...].T, preferred_element_type=f32)
        hs_s[col, :] = jnp.tanh(jnp.dot(xj, sW_ref[...].T,
                                        preferred_element_type=f32))
        fu = jnp.dot(xj, uW_ref[...].T, preferred_element_type=f32)
        fu_s[col, :] = fu.astype(bf16)
        fd = jnp.dot(xj, dW_ref[...].T, preferred_element_type=f32)
        fd_s[col, :] = fd.astype(bf16)
        log2e = 1.4426950408889634
        s1u_s[col, :] = jnp.dot(fu, ua1_ref[...].T,
                                preferred_element_type=f32) * log2e
        s2u_s[0:1, col] = (jnp.dot(fu, ua2_ref[...].T,
                                   preferred_element_type=f32) * log2e).T
        s1d_s[col, :] = jnp.dot(fd, da1_ref[...].T,
                                preferred_element_type=f32) * log2e
        s2d_s[0:1, col] = (jnp.dot(fd, da2_ref[...].T,
                                   preferred_element_type=f32) * log2e).T

    @pl.when(j == 0)
    def _init():
        accp[...] = jnp.zeros_like(accp)
        accu[...] = jnp.zeros_like(accu)
        accd[...] = jnp.zeros_like(accd)
        denu[...] = jnp.zeros_like(denu)
        dend[...] = jnp.zeros_like(dend)

    accp[...] += jnp.dot(l_ref[...], fp_s[col, :], preferred_element_type=f32)

    def _chunk_sum(e):
        s = e[:, 0:128]
        for k in range(1, bn // 128):
            s = s + e[:, k * 128:(k + 1) * 128]
        return s

    tu = s1u_s[row, :] + s2u_s[0:1, col]
    eu = jnp.exp2(jnp.maximum(tu, 0.01 * tu)) * lu_ref[...]
    accu[...] += jnp.dot(eu.astype(bf16), fu_s[col, :],
                         preferred_element_type=f32)
    denu[...] += _chunk_sum(eu)

    td = s1d_s[row, :] + s2d_s[0:1, col]
    ed = jnp.exp2(jnp.maximum(td, 0.01 * td)) * ld_ref[...]
    accd[...] += jnp.dot(ed.astype(bf16), fd_s[col, :],
                         preferred_element_type=f32)
    dend[...] += _chunk_sum(ed)

    @pl.when(j == pl.num_programs(1) - 1)
    def _fini():
        du = jnp.sum(denu[...], axis=1, keepdims=True)
        dd = jnp.sum(dend[...], axis=1, keepdims=True)
        hu = jnp.tanh(accu[...] / jnp.where(du == 0.0, 1.0, du))
        hd = jnp.tanh(accd[...] / jnp.where(dd == 0.0, 1.0, dd))
        out_ref[...] = jnp.tanh(accp[...]) + hs_s[row, :] + hu + hd


def kernel(X, L, L_u, L_d, dimensions, p_W, s_W, u_W, u_a1, u_a2,
           d_W, d_a1, d_a2, *, interpret=False):
    x = X[0]
    l = L[0]
    lu = L_u[0]
    ld = L_d[0]
    n, d_in = x.shape
    d_out = p_W.shape[0]
    f32 = jnp.float32
    bf16 = jnp.bfloat16

    bm = min(512, n)
    bn = min(1024, n)
    grid = (n // bm, n // bn)

    wspec = pl.BlockSpec(p_W.shape, lambda i, j: (0, 0))
    aspec = pl.BlockSpec(u_a1.shape, lambda i, j: (0, 0))
    out = pl.pallas_call(
        functools.partial(_body, bm=bm, bn=bn),
        grid=grid,
        in_specs=[
            pl.BlockSpec((bm, bn), lambda i, j: (i, j)),
            pl.BlockSpec((bm, bn), lambda i, j: (i, j)),
            pl.BlockSpec((bm, bn), lambda i, j: (i, j)),
            pl.BlockSpec((n, d_in), lambda i, j: (0, 0)),
            wspec, wspec, wspec, aspec, aspec, wspec, aspec, aspec,
        ],
        out_specs=pl.BlockSpec((bm, d_out), lambda i, j: (i, 0)),
        out_shape=jax.ShapeDtypeStruct((n, d_out), f32),
        scratch_shapes=[
            pltpu.VMEM((n, d_out), f32),
            pltpu.VMEM((n, d_out), bf16),
            pltpu.VMEM((n, d_out), bf16),
            pltpu.VMEM((n, d_out), f32),
            pltpu.VMEM((n, 1), f32),
            pltpu.VMEM((1, n), f32),
            pltpu.VMEM((n, 1), f32),
            pltpu.VMEM((1, n), f32),
            pltpu.VMEM((bm, d_out), f32),
            pltpu.VMEM((bm, d_out), f32),
            pltpu.VMEM((bm, d_out), f32),
            pltpu.VMEM((bm, 128), f32),
            pltpu.VMEM((bm, 128), f32),
        ],
        compiler_params=pltpu.CompilerParams(
            dimension_semantics=("arbitrary", "arbitrary"),
        ),
        interpret=interpret,
    )
    return out(l, lu, ld, x, p_W, s_W, u_W, u_a1, u_a2, d_W, d_a1, d_a2)


# speedup vs baseline: 1.0559x; 1.0559x over previous
import functools

import jax
import jax.numpy as jnp
from jax.experimental import pallas as pl
from jax.experimental.pallas import tpu as pltpu


def _body(l_ref, lu_ref, ld_ref, x_ref, pW_ref, sW_ref, uW_ref, ua1_ref,
          ua2_ref, dW_ref, da1_ref, da2_ref, out_ref,
          fp_s, fu_s, fd_s, hs_s, s1u_s, s2u_s, s1d_s, s2d_s,
          s1c, accp, accu, accd, denu, dend, *, bm, bn):
    i = pl.program_id(0)
    j = pl.program_id(1)
    f32 = jnp.float32
    bf16 = jnp.bfloat16
    row = pl.ds(i * bm, bm)
    col = pl.ds(j * bn, bn)

    @pl.when(i == 0)
    def _project():
        xj = x_ref[col, :]
        fp_s[col, :] = jnp.dot(xj, pW_ref[...].T,
                               preferred_element_type=f32).astype(bf16)
        hs_s[col, :] = jnp.tanh(jnp.dot(xj, sW_ref[...].T,
                                        preferred_element_type=f32)).astype(bf16)
        fu = jnp.dot(xj, uW_ref[...].T, preferred_element_type=f32)
        fu_s[col, :] = fu.astype(bf16)
        fd = jnp.dot(xj, dW_ref[...].T, preferred_element_type=f32)
        fd_s[col, :] = fd.astype(bf16)
        log2e = 1.4426950408889634
        s1u_s[0:1, col] = (jnp.dot(fu, ua1_ref[...].T,
                                   preferred_element_type=f32) * log2e).T
        s2u_s[0:1, col] = (jnp.dot(fu, ua2_ref[...].T,
                                   preferred_element_type=f32) * log2e).T
        s1d_s[0:1, col] = (jnp.dot(fd, da1_ref[...].T,
                                   preferred_element_type=f32) * log2e).T
        s2d_s[0:1, col] = (jnp.dot(fd, da2_ref[...].T,
                                   preferred_element_type=f32) * log2e).T

    @pl.when(j == 0)
    def _init():
        s1c[:, 0:1] = s1u_s[0:1, row].T
        s1c[:, 1:2] = s1d_s[0:1, row].T
        accp[...] = jnp.zeros_like(accp)
        accu[...] = jnp.zeros_like(accu)
        accd[...] = jnp.zeros_like(accd)
        denu[...] = jnp.zeros_like(denu)
        dend[...] = jnp.zeros_like(dend)

    accp[...] += jnp.dot(l_ref[...].astype(bf16), fp_s[col, :],
                         preferred_element_type=f32)

    def _chunk_sum(e):
        s = e[:, 0:128]
        for k in range(1, bn // 128):
            s = s + e[:, k * 128:(k + 1) * 128]
        return s

    tu = s1c[:, 0:1] + s2u_s[0:1, col]
    eu = jnp.exp2(jnp.maximum(tu, 0.01 * tu)) * lu_ref[...]
    accu[...] += jnp.dot(eu.astype(bf16), fu_s[col, :],
                         preferred_element_type=f32)
    denu[...] += _chunk_sum(eu)

    td = s1c[:, 1:2] + s2d_s[0:1, col]
    ed = jnp.exp2(jnp.maximum(td, 0.01 * td)) * ld_ref[...]
    accd[...] += jnp.dot(ed.astype(bf16), fd_s[col, :],
                         preferred_element_type=f32)
    dend[...] += _chunk_sum(ed)

    @pl.when(j == pl.num_programs(1) - 1)
    def _fini():
        du = jnp.sum(denu[...], axis=1, keepdims=True)
        dd = jnp.sum(dend[...], axis=1, keepdims=True)
        hu = jnp.tanh(accu[...] / jnp.where(du == 0.0, 1.0, du))
        hd = jnp.tanh(accd[...] / jnp.where(dd == 0.0, 1.0, dd))
        out_ref[...] = (jnp.tanh(accp[...]) + hs_s[row, :].astype(jnp.float32)
                        + hu + hd)


def kernel(X, L, L_u, L_d, dimensions, p_W, s_W, u_W, u_a1, u_a2,
           d_W, d_a1, d_a2, *, interpret=False):
    x = X[0]
    l = L[0]
    lu = L_u[0]
    ld = L_d[0]
    n, d_in = x.shape
    d_out = p_W.shape[0]
    f32 = jnp.float32
    bf16 = jnp.bfloat16

    bm = min(1024, n)
    bn = min(2048, n)
    grid = (n // bm, n // bn)

    wspec = pl.BlockSpec(p_W.shape, lambda i, j: (0, 0))
    aspec = pl.BlockSpec(u_a1.shape, lambda i, j: (0, 0))
    out = pl.pallas_call(
        functools.partial(_body, bm=bm, bn=bn),
        grid=grid,
        in_specs=[
            pl.BlockSpec((bm, bn), lambda i, j: (i, j)),
            pl.BlockSpec((bm, bn), lambda i, j: (i, j)),
            pl.BlockSpec((bm, bn), lambda i, j: (i, j)),
            pl.BlockSpec((n, d_in), lambda i, j: (0, 0)),
            wspec, wspec, wspec, aspec, aspec, wspec, aspec, aspec,
        ],
        out_specs=pl.BlockSpec((bm, d_out), lambda i, j: (i, 0)),
        out_shape=jax.ShapeDtypeStruct((n, d_out), f32),
        scratch_shapes=[
            pltpu.VMEM((n, d_out), bf16),
            pltpu.VMEM((n, d_out), bf16),
            pltpu.VMEM((n, d_out), bf16),
            pltpu.VMEM((n, d_out), bf16),
            pltpu.VMEM((1, n), f32),
            pltpu.VMEM((1, n), f32),
            pltpu.VMEM((1, n), f32),
            pltpu.VMEM((1, n), f32),
            pltpu.VMEM((bm, 128), f32),
            pltpu.VMEM((bm, d_out), f32),
            pltpu.VMEM((bm, d_out), f32),
            pltpu.VMEM((bm, d_out), f32),
            pltpu.VMEM((bm, 128), f32),
            pltpu.VMEM((bm, 128), f32),
        ],
        compiler_params=pltpu.CompilerParams(
            dimension_semantics=("arbitrary", "arbitrary"),
            vmem_limit_bytes=100 * 1024 * 1024,
        ),
        interpret=interpret,
    )
    return out(l, lu, ld, x, p_W, s_W, u_W, u_a1, u_a2, d_W, d_a1, d_a2)


# speedup vs baseline: 1.1136x; 1.0546x over previous
import functools

import jax
import jax.numpy as jnp
from jax.experimental import pallas as pl
from jax.experimental.pallas import tpu as pltpu


def _body(l_ref, lu_ref, ld_ref, x_ref, pW_ref, sW_ref, uW_ref, ua1_ref,
          ua2_ref, dW_ref, da1_ref, da2_ref, out_ref,
          fp_s, fu_s, fd_s, hs_s, s1u_s, s2u_s, s1d_s, s2d_s,
          accp, accu, accd, denu, dend, *, bm, bn):
    i = pl.program_id(0)
    j = pl.program_id(1)
    f32 = jnp.float32
    bf16 = jnp.bfloat16
    row = pl.ds(i * bm, bm)
    col = pl.ds(j * bn, bn)

    @pl.when(i == 0)
    def _project():
        xj = x_ref[col, :]
        fp_s[col, :] = jnp.dot(xj, pW_ref[...].T, preferred_element_type=f32)
        hs_s[col, :] = jnp.tanh(jnp.dot(xj, sW_ref[...].T,
                                        preferred_element_type=f32))
        fu = jnp.dot(xj, uW_ref[...].T, preferred_element_type=f32)
        fu_s[col, :] = fu.astype(bf16)
        fd = jnp.dot(xj, dW_ref[...].T, preferred_element_type=f32)
        fd_s[col, :] = fd.astype(bf16)
        log2e = 1.4426950408889634
        s1u_s[col, :] = jnp.dot(fu, ua1_ref[...].T,
                                preferred_element_type=f32) * log2e
        s2u_s[0:1, col] = (jnp.dot(fu, ua2_ref[...].T,
                                   preferred_element_type=f32) * log2e).T
        s1d_s[col, :] = jnp.dot(fd, da1_ref[...].T,
                                preferred_element_type=f32) * log2e
        s2d_s[0:1, col] = (jnp.dot(fd, da2_ref[...].T,
                                   preferred_element_type=f32) * log2e).T

    @pl.when(j == 0)
    def _init():
        accp[...] = jnp.zeros_like(accp)
        accu[...] = jnp.zeros_like(accu)
        accd[...] = jnp.zeros_like(accd)
        denu[...] = jnp.zeros_like(denu)
        dend[...] = jnp.zeros_like(dend)

    accp[...] += jnp.dot(l_ref[...], fp_s[col, :], preferred_element_type=f32)

    def _chunk_sum(e):
        s = e[:, 0:128]
        for k in range(1, bn // 128):
            s = s + e[:, k * 128:(k + 1) * 128]
        return s

    tu = s1u_s[row, :] + s2u_s[0:1, col]
    eu = jnp.exp2(jnp.maximum(tu, 0.01 * tu)) * lu_ref[...]
    accu[...] += jnp.dot(eu.astype(bf16), fu_s[col, :],
                         preferred_element_type=f32)
    denu[...] += _chunk_sum(eu)

    td = s1d_s[row, :] + s2d_s[0:1, col]
    ed = jnp.exp2(jnp.maximum(td, 0.01 * td)) * ld_ref[...]
    accd[...] += jnp.dot(ed.astype(bf16), fd_s[col, :],
                         preferred_element_type=f32)
    dend[...] += _chunk_sum(ed)

    @pl.when(j == pl.num_programs(1) - 1)
    def _fini():
        du = jnp.sum(denu[...], axis=1, keepdims=True)
        dd = jnp.sum(dend[...], axis=1, keepdims=True)
        hu = jnp.tanh(accu[...] / jnp.where(du == 0.0, 1.0, du))
        hd = jnp.tanh(accd[...] / jnp.where(dd == 0.0, 1.0, dd))
        out_ref[...] = jnp.tanh(accp[...]) + hs_s[row, :] + hu + hd


def kernel(X, L, L_u, L_d, dimensions, p_W, s_W, u_W, u_a1, u_a2,
           d_W, d_a1, d_a2, *, interpret=False):
    x = X[0]
    l = L[0]
    lu = L_u[0]
    ld = L_d[0]
    n, d_in = x.shape
    d_out = p_W.shape[0]
    f32 = jnp.float32
    bf16 = jnp.bfloat16

    bm = min(1024, n)
    bn = min(1024, n)
    grid = (n // bm, n // bn)

    wspec = pl.BlockSpec(p_W.shape, lambda i, j: (0, 0))
    aspec = pl.BlockSpec(u_a1.shape, lambda i, j: (0, 0))
    out = pl.pallas_call(
        functools.partial(_body, bm=bm, bn=bn),
        grid=grid,
        in_specs=[
            pl.BlockSpec((bm, bn), lambda i, j: (i, j)),
            pl.BlockSpec((bm, bn), lambda i, j: (i, j)),
            pl.BlockSpec((bm, bn), lambda i, j: (i, j)),
            pl.BlockSpec((n, d_in), lambda i, j: (0, 0)),
            wspec, wspec, wspec, aspec, aspec, wspec, aspec, aspec,
        ],
        out_specs=pl.BlockSpec((bm, d_out), lambda i, j: (i, 0)),
        out_shape=jax.ShapeDtypeStruct((n, d_out), f32),
        scratch_shapes=[
            pltpu.VMEM((n, d_out), f32),
            pltpu.VMEM((n, d_out), bf16),
            pltpu.VMEM((n, d_out), bf16),
            pltpu.VMEM((n, d_out), f32),
            pltpu.VMEM((n, 1), f32),
            pltpu.VMEM((1, n), f32),
            pltpu.VMEM((n, 1), f32),
            pltpu.VMEM((1, n), f32),
            pltpu.VMEM((bm, d_out), f32),
            pltpu.VMEM((bm, d_out), f32),
            pltpu.VMEM((bm, d_out), f32),
            pltpu.VMEM((bm, 128), f32),
            pltpu.VMEM((bm, 128), f32),
        ],
        compiler_params=pltpu.CompilerParams(
            dimension_semantics=("arbitrary", "arbitrary"),
        ),
        interpret=interpret,
    )
    return out(l, lu, ld, x, p_W, s_W, u_W, u_a1, u_a2, d_W, d_a1, d_a2)
